# baseline (speedup 1.0000x reference)
import jax
import jax.numpy as jnp
from jax import lax
from jax.experimental import pallas as pl
from jax.experimental.pallas import tpu as pltpu
from jax.experimental.pallas import tpu_sc as plsc

N_NODES = 10000
D = 128
DP = 144
DB = 160
N_EDGES = 320000

NC = 2
NS = 16
NW = NC * NS
B = 80
CHUNKS = 125
W_PER = B * CHUNKS
E_PAD = W_PER * NW
ROWS_PER_TILE = 632
N_SH = ROWS_PER_TILE * NS
MASK_HI = -65536


def _prep_body(xp_ref, beta_ref, xtb_ref):
    xp = xp_ref[...]
    ss = jnp.sum(xp * xp, axis=1, keepdims=True)
    rn = lax.rsqrt(ss) * jnp.sqrt(beta_ref[...])
    xtb_ref[...] = jnp.concatenate(
        [xp, rn, jnp.zeros((xp.shape[0], DB - D - 1), jnp.float32)], axis=1
    ).astype(jnp.bfloat16)


def _prep(xp, beta2):
    return pl.pallas_call(
        _prep_body,
        out_shape=jax.ShapeDtypeStruct((N_NODES, DB), jnp.bfloat16),
    )(xp, beta2)


def _combine_body(x_ref, num_ref, beta_ref, o_ref):
    eb = jnp.exp(beta_ref[...])
    s = num_ref[0][:N_NODES] + num_ref[1][:N_NODES]
    scale = (1.0 - s[:, D + 1:D + 2]) * eb
    den = s[:, D:D + 1] + scale + 1e-16
    o_ref[...] = (s[:, :D] + scale * x_ref[...]) / den


def _combine(x, num2, beta2):
    return pl.pallas_call(
        _combine_body,
        out_shape=jax.ShapeDtypeStruct((N_NODES, D), jnp.float32),
    )(x, num2, beta2)


def _widen(v32):
    vi = plsc.bitcast(v32, jnp.int32)
    lo = plsc.bitcast(vi << 16, jnp.float32)
    hi = plsc.bitcast(vi & jnp.int32(MASK_HI), jnp.float32)
    return lo, hi


def _sc_body(xtb, epack, zrows, num_out, num_sh,
             xg0, xg1, sbuf, ib0, ib1, rsb, sg0, sg1, sh0, sh1, si0, si1):
    c = lax.axis_index("c")
    s = lax.axis_index("s")
    wid = s * NC + c
    e0 = jnp.where(lax.iota(jnp.int32, 16) == 0, 1.0, 0.0).astype(jnp.float32)
    xg_b = (xg0, xg1)
    ib_b = (ib0, ib1)
    sg_b = (sg0, sg1)
    sh_b = (sh0, sh1)
    si_b = (si0, si1)

    pltpu.sync_copy(zrows, num_sh.at[pl.ds(s * ROWS_PER_TILE, ROWS_PER_TILE), :])
    plsc.subcore_barrier()

    base0 = wid * CHUNKS * 2 * B

    def issue_idx(t_dyn, b):
        base = base0 + t_dyn * (2 * B)
        pltpu.async_copy(epack.at[pl.ds(base, 2 * B)], ib_b[b], si_b[b])

    def wait_idx(b):
        pltpu.make_async_copy(epack.at[pl.ds(0, 2 * B)], ib_b[b], si_b[b]).wait()

    def issue_gathers(b):
        d1 = pltpu.async_copy(
            xtb.at[ib_b[b].at[pl.ds(0, B)]], xg_b[b].at[pl.ds(0, B), :], sg_b[b])
        d2 = pltpu.async_copy(
            xtb.at[ib_b[b].at[pl.ds(B, B)]], xg_b[b].at[pl.ds(B, B), :], sh_b[b])
        return d1, d2

    def compute(b):
        xg = xg_b[b]
        ib = ib_b[b]

        def edge(e, carry):
            eo = e + B
            wr = [_widen(xg[e, pl.ds(32 * g, 32)]) for g in range(4)]
            wc = [_widen(xg[eo, pl.ds(32 * g, 32)]) for g in range(4)]
            p = [wr[g][0] * wc[g][0] for g in range(4)]
            q = [wr[g][1] * wc[g][1] for g in range(4)]
            t0 = (p[0] + p[1]) + (p[2] + p[3])
            t1 = (q[0] + q[1]) + (q[2] + q[3])
            a = t0 + t1
            dot = jnp.sum(a)
            rnr = _widen(xg[e, pl.ds(D, 32)])[0][0]
            rnc = _widen(xg[eo, pl.ds(D, 32)])[0][0]
            logit = dot * rnr * rnc
            wv = jnp.exp(jnp.broadcast_to(logit, (16,)))
            for g in range(4):
                sbuf[e, pl.ds(32 * g, 16)] = wc[g][0] * wv
                sbuf[e, pl.ds(32 * g + 16, 16)] = wc[g][1] * wv
            sbuf[e, pl.ds(D, 16)] = wv * e0
            return carry

        lax.fori_loop(0, B, edge, 0, unroll=8)

        iota16 = lax.iota(jnp.int32, 16)
        c129 = jnp.full((16,), D + 1, jnp.int32)

        def group(g, carry):
            rb = g * 16
            riv = ib[pl.ds(rb, 16)]
            civ = ib[pl.ds(B + rb, 16)]
            selfv = jnp.where(riv == civ, 1.0, 0.0).astype(jnp.float32)
            plsc.store_scatter(sbuf, [iota16 + rb, c129], selfv)
            return carry

        lax.fori_loop(0, B // 16, group, 0)

    def slot(t_next, b):
        nb = 1 - b
        wait_idx(nb)
        d1, d2 = issue_gathers(nb)
        for k in range(B // 16):
            rsb[pl.ds(16 * k, 16)] = ib_b[b][pl.ds(16 * k, 16)]
        compute(b)
        issue_idx(jnp.minimum(t_next + 1, CHUNKS - 1), b)
        pltpu.sync_copy(sbuf, num_sh.at[rsb], add=True)
        d1.wait()
        d2.wait()

    issue_idx(jnp.int32(0), 0)
    wait_idx(0)
    d1, d2 = issue_gathers(0)
    d1.wait()
    d2.wait()
    issue_idx(jnp.int32(1), 1)

    def body(i, carry):
        t = i * 2
        slot(t + 1, 0)
        slot(jnp.minimum(t + 2, CHUNKS - 1), 1)
        return carry

    lax.fori_loop(0, CHUNKS // 2, body, 0)
    slot(jnp.int32(CHUNKS - 1), 0)
    wait_idx(0)

    plsc.subcore_barrier()
    sl = pl.ds(s * ROWS_PER_TILE, ROWS_PER_TILE)
    pltpu.sync_copy(num_sh.at[sl, :], num_out.at[c, sl, :])


def _sc_edges(xtb, epack, zrows):
    mesh = plsc.VectorSubcoreMesh(
        core_axis_name="c", subcore_axis_name="s", num_cores=NC, num_subcores=NS
    )
    f = pl.kernel(
        _sc_body,
        out_type=jax.ShapeDtypeStruct((NC, N_SH, DP), jnp.float32),
        mesh=mesh,
        compiler_params=pltpu.CompilerParams(
            needs_layout_passes=False, use_tc_tiling_on_sc=False
        ),
        scratch_types=[
            pltpu.VMEM_SHARED((N_SH, DP), jnp.float32),
            pltpu.VMEM((2 * B, DB), jnp.bfloat16),
            pltpu.VMEM((2 * B, DB), jnp.bfloat16),
            pltpu.VMEM((B, DP), jnp.float32),
            pltpu.VMEM((2 * B,), jnp.int32),
            pltpu.VMEM((2 * B,), jnp.int32),
            pltpu.VMEM((B,), jnp.int32),
            pltpu.SemaphoreType.DMA,
            pltpu.SemaphoreType.DMA,
            pltpu.SemaphoreType.DMA,
            pltpu.SemaphoreType.DMA,
            pltpu.SemaphoreType.DMA,
            pltpu.SemaphoreType.DMA,
        ],
    )
    return f(xtb, epack, zrows)


@jax.jit
def kernel(x, edge_index, beta):
    x = x.astype(jnp.float32)
    beta2 = beta.reshape(1, 1).astype(jnp.float32)
    row = edge_index[0].astype(jnp.int32)
    col = edge_index[1].astype(jnp.int32)
    pad = E_PAD - row.shape[0]
    zed = jnp.zeros((pad,), jnp.int32)
    rowp = jnp.concatenate([row, zed]).reshape(NW, CHUNKS, B)
    colp = jnp.concatenate([col, zed]).reshape(NW, CHUNKS, B)
    epack = jnp.concatenate([rowp, colp], axis=2).reshape(-1)
    xp = x.reshape(N_NODES, 4, 2, 16).transpose(0, 1, 3, 2).reshape(N_NODES, D)
    xtb = _prep(xp, beta2)
    zrows = jnp.zeros((ROWS_PER_TILE, DP), jnp.float32)
    num2 = _sc_edges(xtb, epack, zrows)
    return _combine(x, num2, beta2)


# speedup vs baseline: 1.0009x; 1.0009x over previous
import jax
import jax.numpy as jnp
from jax import lax
from jax.experimental import pallas as pl
from jax.experimental.pallas import tpu as pltpu
from jax.experimental.pallas import tpu_sc as plsc

N_NODES = 10000
D = 128
DP = 144
DB = 160
N_EDGES = 320000

NC = 2
NS = 16
NW = NC * NS
B = 80
CHUNKS = 125
W_PER = B * CHUNKS
E_PAD = W_PER * NW
ROWS_PER_TILE = 632
N_SH = ROWS_PER_TILE * NS
MASK_HI = -65536


def _prep_body(xp_ref, beta_ref, xtb_ref):
    xp = xp_ref[...]
    ss = jnp.sum(xp * xp, axis=1, keepdims=True)
    rn = lax.rsqrt(ss) * jnp.sqrt(beta_ref[...])
    xtb_ref[...] = jnp.concatenate(
        [xp, rn, jnp.zeros((xp.shape[0], DB - D - 1), jnp.float32)], axis=1
    ).astype(jnp.bfloat16)


def _prep(xp, beta2):
    return pl.pallas_call(
        _prep_body,
        out_shape=jax.ShapeDtypeStruct((N_NODES, DB), jnp.bfloat16),
    )(xp, beta2)


def _combine_body(x_ref, num_ref, beta_ref, o_ref):
    eb = jnp.exp(beta_ref[...])
    s = num_ref[0][:N_NODES] + num_ref[1][:N_NODES]
    scale = (1.0 - s[:, D + 1:D + 2]) * eb
    den = s[:, D:D + 1] + scale + 1e-16
    o_ref[...] = (s[:, :D] + scale * x_ref[...]) / den


def _combine(x, num2, beta2):
    return pl.pallas_call(
        _combine_body,
        out_shape=jax.ShapeDtypeStruct((N_NODES, D), jnp.float32),
    )(x, num2, beta2)


def _widen(v32):
    vi = plsc.bitcast(v32, jnp.int32)
    lo = plsc.bitcast(vi << 16, jnp.float32)
    hi = plsc.bitcast(vi & jnp.int32(MASK_HI), jnp.float32)
    return lo, hi


def _sc_body(xtb, epack, zrows, num_out, num_sh,
             xg0, xg1, sbuf, ib0, ib1, rsb, sg0, sg1, sh0, sh1, si0, si1):
    c = lax.axis_index("c")
    s = lax.axis_index("s")
    wid = s * NC + c
    e0 = jnp.where(lax.iota(jnp.int32, 16) == 0, 1.0, 0.0).astype(jnp.float32)
    xg_b = (xg0, xg1)
    ib_b = (ib0, ib1)
    sg_b = (sg0, sg1)
    sh_b = (sh0, sh1)
    si_b = (si0, si1)

    pltpu.sync_copy(zrows, num_sh.at[pl.ds(s * ROWS_PER_TILE, ROWS_PER_TILE), :])
    plsc.subcore_barrier()

    base0 = wid * CHUNKS * 2 * B

    def issue_idx(t_dyn, b):
        base = base0 + t_dyn * (2 * B)
        pltpu.async_copy(epack.at[pl.ds(base, 2 * B)], ib_b[b], si_b[b])

    def wait_idx(b):
        pltpu.make_async_copy(epack.at[pl.ds(0, 2 * B)], ib_b[b], si_b[b]).wait()

    def issue_gathers(b):
        d1 = pltpu.async_copy(
            xtb.at[ib_b[b].at[pl.ds(0, B)]], xg_b[b].at[pl.ds(0, B), :], sg_b[b])
        d2 = pltpu.async_copy(
            xtb.at[ib_b[b].at[pl.ds(B, B)]], xg_b[b].at[pl.ds(B, B), :], sh_b[b])
        return d1, d2

    def compute(b):
        xg = xg_b[b]
        ib = ib_b[b]

        def edge(e, carry):
            eo = e + B
            wr = [_widen(xg[e, pl.ds(32 * g, 32)]) for g in range(4)]
            wc = [_widen(xg[eo, pl.ds(32 * g, 32)]) for g in range(4)]
            p = [wr[g][0] * wc[g][0] for g in range(4)]
            q = [wr[g][1] * wc[g][1] for g in range(4)]
            t0 = (p[0] + p[1]) + (p[2] + p[3])
            t1 = (q[0] + q[1]) + (q[2] + q[3])
            a = t0 + t1
            dot = jnp.sum(a)
            rnr = _widen(xg[e, pl.ds(D, 32)])[0][0]
            rnc = _widen(xg[eo, pl.ds(D, 32)])[0][0]
            logit = dot * rnr * rnc
            wv = jnp.exp(jnp.broadcast_to(logit, (16,)))
            for g in range(4):
                sbuf[e, pl.ds(32 * g, 16)] = wc[g][0] * wv
                sbuf[e, pl.ds(32 * g + 16, 16)] = wc[g][1] * wv
            sbuf[e, pl.ds(D, 16)] = wv * e0
            return carry

        lax.fori_loop(0, B, edge, 0, unroll=4)

        iota16 = lax.iota(jnp.int32, 16)
        c129 = jnp.full((16,), D + 1, jnp.int32)

        def group(g, carry):
            rb = g * 16
            riv = ib[pl.ds(rb, 16)]
            civ = ib[pl.ds(B + rb, 16)]
            selfv = jnp.where(riv == civ, 1.0, 0.0).astype(jnp.float32)
            plsc.store_scatter(sbuf, [iota16 + rb, c129], selfv)
            return carry

        lax.fori_loop(0, B // 16, group, 0)

    def slot(t_next, b):
        nb = 1 - b
        wait_idx(nb)
        d1, d2 = issue_gathers(nb)
        for k in range(B // 16):
            rsb[pl.ds(16 * k, 16)] = ib_b[b][pl.ds(16 * k, 16)]
        compute(b)
        issue_idx(jnp.minimum(t_next + 1, CHUNKS - 1), b)
        pltpu.sync_copy(sbuf, num_sh.at[rsb], add=True)
        d1.wait()
        d2.wait()

    issue_idx(jnp.int32(0), 0)
    wait_idx(0)
    d1, d2 = issue_gathers(0)
    d1.wait()
    d2.wait()
    issue_idx(jnp.int32(1), 1)

    def body(i, carry):
        t = i * 2
        slot(t + 1, 0)
        slot(jnp.minimum(t + 2, CHUNKS - 1), 1)
        return carry

    lax.fori_loop(0, CHUNKS // 2, body, 0)
    slot(jnp.int32(CHUNKS - 1), 0)
    wait_idx(0)

    plsc.subcore_barrier()
    sl = pl.ds(s * ROWS_PER_TILE, ROWS_PER_TILE)
    pltpu.sync_copy(num_sh.at[sl, :], num_out.at[c, sl, :])


def _sc_edges(xtb, epack, zrows):
    mesh = plsc.VectorSubcoreMesh(
        core_axis_name="c", subcore_axis_name="s", num_cores=NC, num_subcores=NS
    )
    f = pl.kernel(
        _sc_body,
        out_type=jax.ShapeDtypeStruct((NC, N_SH, DP), jnp.float32),
        mesh=mesh,
        compiler_params=pltpu.CompilerParams(
            needs_layout_passes=False, use_tc_tiling_on_sc=False
        ),
        scratch_types=[
            pltpu.VMEM_SHARED((N_SH, DP), jnp.float32),
            pltpu.VMEM((2 * B, DB), jnp.bfloat16),
            pltpu.VMEM((2 * B, DB), jnp.bfloat16),
            pltpu.VMEM((B, DP), jnp.float32),
            pltpu.VMEM((2 * B,), jnp.int32),
            pltpu.VMEM((2 * B,), jnp.int32),
            pltpu.VMEM((B,), jnp.int32),
            pltpu.SemaphoreType.DMA,
            pltpu.SemaphoreType.DMA,
            pltpu.SemaphoreType.DMA,
            pltpu.SemaphoreType.DMA,
            pltpu.SemaphoreType.DMA,
            pltpu.SemaphoreType.DMA,
        ],
    )
    return f(xtb, epack, zrows)


@jax.jit
def kernel(x, edge_index, beta):
    x = x.astype(jnp.float32)
    beta2 = beta.reshape(1, 1).astype(jnp.float32)
    row = edge_index[0].astype(jnp.int32)
    col = edge_index[1].astype(jnp.int32)
    pad = E_PAD - row.shape[0]
    zed = jnp.zeros((pad,), jnp.int32)
    rowp = jnp.concatenate([row, zed]).reshape(NW, CHUNKS, B)
    colp = jnp.concatenate([col, zed]).reshape(NW, CHUNKS, B)
    epack = jnp.concatenate([rowp, colp], axis=2).reshape(-1)
    xp = x.reshape(N_NODES, 4, 2, 16).transpose(0, 1, 3, 2).reshape(N_NODES, D)
    xtb = _prep(xp, beta2)
    zrows = jnp.zeros((ROWS_PER_TILE, DP), jnp.float32)
    num2 = _sc_edges(xtb, epack, zrows)
    return _combine(x, num2, beta2)
